# baseline (speedup 1.0000x reference)
import functools

import jax
import jax.numpy as jnp
from jax.experimental import pallas as pl
from jax.experimental.pallas import tpu as pltpu
from jax.experimental.pallas import tpu_sc as plsc

_E = 8
_TOPK = 2
_TM = 128
_RT = 512
_SM = 512
_SK = 512

_NC = 2
_NS = 16
_NW = _NC * _NS


def _silu(x):
    return x / (1.0 + jnp.exp(-x))


def _router_body(x_ref, rw_ref, w_ref, idx_ref):
    x = x_ref[...]
    rw = rw_ref[...]
    logits = jax.lax.dot_general(x, rw, (((1,), (1,)), ((), ())),
                                 preferred_element_type=jnp.float32)
    m = jnp.max(logits, axis=-1, keepdims=True)
    p = jnp.exp(logits - m)
    p = p / jnp.sum(p, axis=-1, keepdims=True)
    iota = jax.lax.broadcasted_iota(jnp.int32, p.shape, 1)
    m1 = jnp.max(p, axis=-1, keepdims=True)
    i1 = jnp.min(jnp.where(p == m1, iota, _E), axis=-1, keepdims=True)
    pm = jnp.where(iota == i1, -1.0, p)
    m2 = jnp.max(pm, axis=-1, keepdims=True)
    i2 = jnp.min(jnp.where(pm == m2, iota, _E), axis=-1, keepdims=True)
    s = jnp.maximum(m1 + m2, 1e-9)
    w_ref[...] = jnp.concatenate([m1 / s, m2 / s], axis=-1)
    idx_ref[...] = jnp.concatenate([i1, i2], axis=-1).astype(jnp.int32)


def _make_sc_dispatch(NA, H, C, M_PAD):
    rows_per_w = NA // _NW
    n_chunks = rows_per_w // C
    mesh = plsc.VectorSubcoreMesh(core_axis_name="c", subcore_axis_name="s")

    @functools.partial(
        pl.kernel,
        out_type=jax.ShapeDtypeStruct((M_PAD, H), jnp.float32),
        mesh=mesh,
        scratch_types=[
            pltpu.VMEM((C,), jnp.int32),
            pltpu.VMEM((C,), jnp.int32),
            pltpu.VMEM((C, H), jnp.float32),
            pltpu.SemaphoreType.DMA,
            pltpu.SemaphoreType.DMA,
        ],
    )
    def k(tok_hbm, pos_hbm, src_hbm, out_hbm, tok_v, pos_v, rows_v, s1, s2):
        wid = jax.lax.axis_index("s") * _NC + jax.lax.axis_index("c")
        base = wid * rows_per_w

        def body(j, carry):
            off = base + j * C
            pltpu.sync_copy(tok_hbm.at[pl.ds(off, C)], tok_v)
            pltpu.sync_copy(pos_hbm.at[pl.ds(off, C)], pos_v)
            pltpu.async_copy(src_hbm.at[tok_v], rows_v, s1).wait()
            pltpu.async_copy(rows_v, out_hbm.at[pos_v], s2).wait()
            return carry

        jax.lax.fori_loop(0, n_chunks, body, 0)

    return k


def _make_sc_combine_gather(N, H, C):
    rows_per_w = N // _NW
    n_chunks = rows_per_w // C
    mesh = plsc.VectorSubcoreMesh(core_axis_name="c", subcore_axis_name="s")

    @functools.partial(
        pl.kernel,
        out_type=[jax.ShapeDtypeStruct((N, H), jnp.float32),
                  jax.ShapeDtypeStruct((N, H), jnp.float32)],
        mesh=mesh,
        scratch_types=[
            pltpu.VMEM((C,), jnp.int32),
            pltpu.VMEM((C, H), jnp.float32),
            pltpu.SemaphoreType.DMA,
        ],
    )
    def k(p0_hbm, p1_hbm, y_hbm, o0_hbm, o1_hbm, idx_v, rows_v, sem):
        wid = jax.lax.axis_index("s") * _NC + jax.lax.axis_index("c")
        base = wid * rows_per_w

        def body(j, carry):
            off = base + j * C
            pltpu.sync_copy(p0_hbm.at[pl.ds(off, C)], idx_v)
            pltpu.async_copy(y_hbm.at[idx_v], rows_v, sem).wait()
            pltpu.sync_copy(rows_v, o0_hbm.at[pl.ds(off, C)])
            pltpu.sync_copy(p1_hbm.at[pl.ds(off, C)], idx_v)
            pltpu.async_copy(y_hbm.at[idx_v], rows_v, sem).wait()
            pltpu.sync_copy(rows_v, o1_hbm.at[pl.ds(off, C)])
            return carry

        jax.lax.fori_loop(0, n_chunks, body, 0)

    return k


def _gmm_body(e_ref, x_ref, wg_ref, wu_ref, wd_ref, y_ref):
    del e_ref
    x = x_ref[...].astype(jnp.bfloat16)
    g = jax.lax.dot_general(x, wg_ref[0], (((1,), (1,)), ((), ())),
                            preferred_element_type=jnp.float32)
    u = jax.lax.dot_general(x, wu_ref[0], (((1,), (1,)), ((), ())),
                            preferred_element_type=jnp.float32)
    a = (_silu(g) * u).astype(jnp.bfloat16)
    y_ref[...] = jax.lax.dot_general(a, wd_ref[0], (((1,), (1,)), ((), ())),
                                     preferred_element_type=jnp.float32)


def _shared_body(n_k, x_ref, sg_ref, su_ref, sd_ref, gw_ref, o_ref):
    k = pl.program_id(1)
    x = x_ref[...].astype(jnp.bfloat16)
    g = jax.lax.dot_general(x, sg_ref[...], (((1,), (1,)), ((), ())),
                            preferred_element_type=jnp.float32)
    u = jax.lax.dot_general(x, su_ref[...], (((1,), (1,)), ((), ())),
                            preferred_element_type=jnp.float32)
    a = (_silu(g) * u).astype(jnp.bfloat16)
    part = jax.lax.dot_general(a, sd_ref[...], (((1,), (1,)), ((), ())),
                               preferred_element_type=jnp.float32)

    @pl.when(k == 0)
    def _():
        o_ref[...] = part

    @pl.when(k != 0)
    def _():
        o_ref[...] += part

    @pl.when(k == n_k - 1)
    def _():
        gl = jax.lax.dot_general(x_ref[...], gw_ref[...],
                                 (((1,), (1,)), ((), ())),
                                 preferred_element_type=jnp.float32)
        o_ref[...] *= 1.0 / (1.0 + jnp.exp(-gl))


def _combine_body(y0_ref, y1_ref, w_ref, sh_ref, o_ref):
    w = w_ref[...]
    o_ref[...] = (y0_ref[...] * w[:, 0:1] + y1_ref[...] * w[:, 1:2]
                  + sh_ref[...])


def kernel(hidden_states, router_w, Wg, Wu, Wd, Sg, Su, Sd, gate_w):
    orig_shape = hidden_states.shape
    H = orig_shape[-1]
    h2 = hidden_states.reshape(-1, H)
    Wg = Wg.astype(jnp.bfloat16)
    Wu = Wu.astype(jnp.bfloat16)
    Wd = Wd.astype(jnp.bfloat16)
    Sg = Sg.astype(jnp.bfloat16)
    Su = Su.astype(jnp.bfloat16)
    Sd = Sd.astype(jnp.bfloat16)
    N = h2.shape[0]
    INTER = Wg.shape[1]
    SH = Sg.shape[0]
    NA = N * _TOPK
    M_PAD = NA + _E * _TM
    n_tiles = M_PAD // _TM

    w, idx = pl.pallas_call(
        _router_body,
        grid=(N // _RT,),
        in_specs=[pl.BlockSpec((_RT, H), lambda i: (i, 0)),
                  pl.BlockSpec((_E, H), lambda i: (0, 0))],
        out_specs=[pl.BlockSpec((_RT, _TOPK), lambda i: (i, 0)),
                   pl.BlockSpec((_RT, _TOPK), lambda i: (i, 0))],
        out_shape=[jax.ShapeDtypeStruct((N, _TOPK), jnp.float32),
                   jax.ShapeDtypeStruct((N, _TOPK), jnp.int32)],
    )(h2, router_w)

    a = idx.reshape(-1)
    oh = (a[:, None] == jnp.arange(_E, dtype=jnp.int32)[None, :]).astype(jnp.int32)
    ranks = jnp.cumsum(oh, axis=0) - oh
    counts = jnp.sum(oh, axis=0)
    padded = ((counts + _TM - 1) // _TM) * _TM
    base = jnp.concatenate([jnp.zeros((1,), jnp.int32),
                            jnp.cumsum(padded)[:-1].astype(jnp.int32)])
    pos_flat = base[a] + jnp.take_along_axis(ranks, a[:, None], axis=1)[:, 0]
    tile_start = jnp.arange(n_tiles, dtype=jnp.int32) * _TM
    expert_of_tile = jnp.clip(
        jnp.searchsorted(base, tile_start, side="right") - 1, 0, _E - 1
    ).astype(jnp.int32)
    pos = pos_flat.reshape(N, _TOPK)
    tok_src = jnp.arange(NA, dtype=jnp.int32) // _TOPK

    x_sorted = _make_sc_dispatch(NA, H, 16, M_PAD)(tok_src, pos_flat, h2)

    n_k = SH // _SK
    shared_g = pl.pallas_call(
        functools.partial(_shared_body, n_k),
        grid=(N // _SM, n_k),
        in_specs=[
            pl.BlockSpec((_SM, H), lambda i, k: (i, 0)),
            pl.BlockSpec((_SK, H), lambda i, k: (k, 0)),
            pl.BlockSpec((_SK, H), lambda i, k: (k, 0)),
            pl.BlockSpec((H, _SK), lambda i, k: (0, k)),
            pl.BlockSpec((1, H), lambda i, k: (0, 0)),
        ],
        out_specs=pl.BlockSpec((_SM, H), lambda i, k: (i, 0)),
        out_shape=jax.ShapeDtypeStruct((N, H), jnp.float32),
    )(h2, Sg, Su, Sd, gate_w)

    y_sorted = pl.pallas_call(
        _gmm_body,
        grid_spec=pltpu.PrefetchScalarGridSpec(
            num_scalar_prefetch=1,
            grid=(n_tiles,),
            in_specs=[
                pl.BlockSpec((_TM, H), lambda i, e: (i, 0)),
                pl.BlockSpec((1, INTER, H), lambda i, e: (e[i], 0, 0)),
                pl.BlockSpec((1, INTER, H), lambda i, e: (e[i], 0, 0)),
                pl.BlockSpec((1, H, INTER), lambda i, e: (e[i], 0, 0)),
            ],
            out_specs=pl.BlockSpec((_TM, H), lambda i, e: (i, 0)),
        ),
        out_shape=jax.ShapeDtypeStruct((M_PAD, H), jnp.float32),
    )(expert_of_tile, x_sorted, Wg, Wu, Wd)

    y0g, y1g = _make_sc_combine_gather(N, H, 16)(pos[:, 0], pos[:, 1],
                                                 y_sorted)
    out2 = pl.pallas_call(
        _combine_body,
        grid=(N // 512,),
        in_specs=[pl.BlockSpec((512, H), lambda i: (i, 0)),
                  pl.BlockSpec((512, H), lambda i: (i, 0)),
                  pl.BlockSpec((512, _TOPK), lambda i: (i, 0)),
                  pl.BlockSpec((512, H), lambda i: (i, 0))],
        out_specs=pl.BlockSpec((512, H), lambda i: (i, 0)),
        out_shape=jax.ShapeDtypeStruct((N, H), jnp.float32),
    )(y0g, y1g, w, shared_g)

    return out2.reshape(orig_shape)


# speedup vs baseline: 1.1047x; 1.1047x over previous
import functools

import jax
import jax.numpy as jnp
from jax.experimental import pallas as pl
from jax.experimental.pallas import tpu as pltpu
from jax.experimental.pallas import tpu_sc as plsc

_E = 8
_TOPK = 2
_TM = 128
_RT = 512
_SM = 512
_SK = 512

_NC = 2
_NS = 16
_NW = _NC * _NS


def _silu(x):
    return x / (1.0 + jnp.exp(-x))


def _router_body(x_ref, rw_ref, w_ref, idx_ref):
    x = x_ref[...]
    rw = rw_ref[...]
    logits = jax.lax.dot_general(x, rw, (((1,), (1,)), ((), ())),
                                 preferred_element_type=jnp.float32)
    m = jnp.max(logits, axis=-1, keepdims=True)
    p = jnp.exp(logits - m)
    p = p / jnp.sum(p, axis=-1, keepdims=True)
    iota = jax.lax.broadcasted_iota(jnp.int32, p.shape, 1)
    m1 = jnp.max(p, axis=-1, keepdims=True)
    i1 = jnp.min(jnp.where(p == m1, iota, _E), axis=-1, keepdims=True)
    pm = jnp.where(iota == i1, -1.0, p)
    m2 = jnp.max(pm, axis=-1, keepdims=True)
    i2 = jnp.min(jnp.where(pm == m2, iota, _E), axis=-1, keepdims=True)
    s = jnp.maximum(m1 + m2, 1e-9)
    w_ref[...] = jnp.concatenate([m1 / s, m2 / s], axis=-1)
    idx_ref[...] = jnp.concatenate([i1, i2], axis=-1).astype(jnp.int32)


def _make_sc_dispatch(NA, H, C, M_PAD):
    rows_per_w = NA // _NW
    n_chunks = rows_per_w // C
    mesh = plsc.VectorSubcoreMesh(core_axis_name="c", subcore_axis_name="s")

    @functools.partial(
        pl.kernel,
        out_type=jax.ShapeDtypeStruct((M_PAD, H), jnp.float32),
        mesh=mesh,
        scratch_types=[
            pltpu.VMEM((C,), jnp.int32),
            pltpu.VMEM((C,), jnp.int32),
            pltpu.VMEM((C, H), jnp.float32),
            pltpu.SemaphoreType.DMA,
            pltpu.SemaphoreType.DMA,
        ],
    )
    def k(tok_hbm, pos_hbm, src_hbm, out_hbm, tok_v, pos_v, rows_v, s1, s2):
        wid = jax.lax.axis_index("s") * _NC + jax.lax.axis_index("c")
        base = wid * rows_per_w

        def body(j, carry):
            off = base + j * C
            pltpu.sync_copy(tok_hbm.at[pl.ds(off, C)], tok_v)
            pltpu.sync_copy(pos_hbm.at[pl.ds(off, C)], pos_v)
            pltpu.async_copy(src_hbm.at[tok_v], rows_v, s1).wait()
            pltpu.async_copy(rows_v, out_hbm.at[pos_v], s2).wait()
            return carry

        jax.lax.fori_loop(0, n_chunks, body, 0)

    return k


def _make_sc_combine_gather(N, H, C):
    rows_per_w = N // _NW
    n_chunks = rows_per_w // C
    mesh = plsc.VectorSubcoreMesh(core_axis_name="c", subcore_axis_name="s")

    @functools.partial(
        pl.kernel,
        out_type=[jax.ShapeDtypeStruct((N, H), jnp.float32),
                  jax.ShapeDtypeStruct((N, H), jnp.float32)],
        mesh=mesh,
        scratch_types=[
            pltpu.VMEM((C,), jnp.int32),
            pltpu.VMEM((C, H), jnp.float32),
            pltpu.SemaphoreType.DMA,
        ],
    )
    def k(p0_hbm, p1_hbm, y_hbm, o0_hbm, o1_hbm, idx_v, rows_v, sem):
        wid = jax.lax.axis_index("s") * _NC + jax.lax.axis_index("c")
        base = wid * rows_per_w

        def body(j, carry):
            off = base + j * C
            pltpu.sync_copy(p0_hbm.at[pl.ds(off, C)], idx_v)
            pltpu.async_copy(y_hbm.at[idx_v], rows_v, sem).wait()
            pltpu.sync_copy(rows_v, o0_hbm.at[pl.ds(off, C)])
            pltpu.sync_copy(p1_hbm.at[pl.ds(off, C)], idx_v)
            pltpu.async_copy(y_hbm.at[idx_v], rows_v, sem).wait()
            pltpu.sync_copy(rows_v, o1_hbm.at[pl.ds(off, C)])
            return carry

        jax.lax.fori_loop(0, n_chunks, body, 0)

    return k


def _gmm_body(e_ref, x_ref, wg_ref, wu_ref, wd_ref, y_ref):
    del e_ref
    x = x_ref[...].astype(jnp.bfloat16)
    g = jax.lax.dot_general(x, wg_ref[0].astype(jnp.bfloat16),
                            (((1,), (1,)), ((), ())),
                            preferred_element_type=jnp.float32)
    u = jax.lax.dot_general(x, wu_ref[0].astype(jnp.bfloat16),
                            (((1,), (1,)), ((), ())),
                            preferred_element_type=jnp.float32)
    a = (_silu(g) * u).astype(jnp.bfloat16)
    y_ref[...] = jax.lax.dot_general(a, wd_ref[0].astype(jnp.bfloat16),
                                     (((1,), (1,)), ((), ())),
                                     preferred_element_type=jnp.float32)


def _shared_body(n_k, x_ref, sg_ref, su_ref, sd_ref, gw_ref, o_ref):
    k = pl.program_id(1)
    x = x_ref[...].astype(jnp.bfloat16)
    g = jax.lax.dot_general(x, sg_ref[...].astype(jnp.bfloat16),
                            (((1,), (1,)), ((), ())),
                            preferred_element_type=jnp.float32)
    u = jax.lax.dot_general(x, su_ref[...].astype(jnp.bfloat16),
                            (((1,), (1,)), ((), ())),
                            preferred_element_type=jnp.float32)
    a = (_silu(g) * u).astype(jnp.bfloat16)
    part = jax.lax.dot_general(a, sd_ref[...].astype(jnp.bfloat16),
                               (((1,), (1,)), ((), ())),
                               preferred_element_type=jnp.float32)

    @pl.when(k == 0)
    def _():
        o_ref[...] = part

    @pl.when(k != 0)
    def _():
        o_ref[...] += part

    @pl.when(k == n_k - 1)
    def _():
        gl = jax.lax.dot_general(x_ref[...], gw_ref[...],
                                 (((1,), (1,)), ((), ())),
                                 preferred_element_type=jnp.float32)
        o_ref[...] *= 1.0 / (1.0 + jnp.exp(-gl))


def _combine_body(y0_ref, y1_ref, w_ref, sh_ref, o_ref):
    w = w_ref[...]
    o_ref[...] = (y0_ref[...] * w[:, 0:1] + y1_ref[...] * w[:, 1:2]
                  + sh_ref[...])


def kernel(hidden_states, router_w, Wg, Wu, Wd, Sg, Su, Sd, gate_w):
    orig_shape = hidden_states.shape
    H = orig_shape[-1]
    h2 = hidden_states.reshape(-1, H)
    N = h2.shape[0]
    INTER = Wg.shape[1]
    SH = Sg.shape[0]
    NA = N * _TOPK
    M_PAD = NA + _E * _TM
    n_tiles = M_PAD // _TM

    w, idx = pl.pallas_call(
        _router_body,
        grid=(N // _RT,),
        in_specs=[pl.BlockSpec((_RT, H), lambda i: (i, 0)),
                  pl.BlockSpec((_E, H), lambda i: (0, 0))],
        out_specs=[pl.BlockSpec((_RT, _TOPK), lambda i: (i, 0)),
                   pl.BlockSpec((_RT, _TOPK), lambda i: (i, 0))],
        out_shape=[jax.ShapeDtypeStruct((N, _TOPK), jnp.float32),
                   jax.ShapeDtypeStruct((N, _TOPK), jnp.int32)],
    )(h2, router_w)

    a = idx.reshape(-1)
    oh = (a[:, None] == jnp.arange(_E, dtype=jnp.int32)[None, :]).astype(jnp.int32)
    ranks = jnp.cumsum(oh, axis=0) - oh
    counts = jnp.sum(oh, axis=0)
    padded = ((counts + _TM - 1) // _TM) * _TM
    base = jnp.concatenate([jnp.zeros((1,), jnp.int32),
                            jnp.cumsum(padded)[:-1].astype(jnp.int32)])
    pos_flat = base[a] + jnp.take_along_axis(ranks, a[:, None], axis=1)[:, 0]
    tile_start = jnp.arange(n_tiles, dtype=jnp.int32) * _TM
    expert_of_tile = jnp.clip(
        jnp.searchsorted(base, tile_start, side="right") - 1, 0, _E - 1
    ).astype(jnp.int32)
    pos = pos_flat.reshape(N, _TOPK)
    tok_src = jnp.arange(NA, dtype=jnp.int32) // _TOPK

    x_sorted = _make_sc_dispatch(NA, H, 16, M_PAD)(tok_src, pos_flat, h2)

    n_k = SH // _SK
    shared_g = pl.pallas_call(
        functools.partial(_shared_body, n_k),
        grid=(N // _SM, n_k),
        in_specs=[
            pl.BlockSpec((_SM, H), lambda i, k: (i, 0)),
            pl.BlockSpec((_SK, H), lambda i, k: (k, 0)),
            pl.BlockSpec((_SK, H), lambda i, k: (k, 0)),
            pl.BlockSpec((H, _SK), lambda i, k: (0, k)),
            pl.BlockSpec((1, H), lambda i, k: (0, 0)),
        ],
        out_specs=pl.BlockSpec((_SM, H), lambda i, k: (i, 0)),
        out_shape=jax.ShapeDtypeStruct((N, H), jnp.float32),
    )(h2, Sg, Su, Sd, gate_w)

    y_sorted = pl.pallas_call(
        _gmm_body,
        grid_spec=pltpu.PrefetchScalarGridSpec(
            num_scalar_prefetch=1,
            grid=(n_tiles,),
            in_specs=[
                pl.BlockSpec((_TM, H), lambda i, e: (i, 0)),
                pl.BlockSpec((1, INTER, H), lambda i, e: (e[i], 0, 0)),
                pl.BlockSpec((1, INTER, H), lambda i, e: (e[i], 0, 0)),
                pl.BlockSpec((1, H, INTER), lambda i, e: (e[i], 0, 0)),
            ],
            out_specs=pl.BlockSpec((_TM, H), lambda i, e: (i, 0)),
        ),
        out_shape=jax.ShapeDtypeStruct((M_PAD, H), jnp.float32),
    )(expert_of_tile, x_sorted, Wg, Wu, Wd)

    y0g, y1g = _make_sc_combine_gather(N, H, 16)(pos[:, 0], pos[:, 1],
                                                 y_sorted)
    out2 = pl.pallas_call(
        _combine_body,
        grid=(N // 512,),
        in_specs=[pl.BlockSpec((512, H), lambda i: (i, 0)),
                  pl.BlockSpec((512, H), lambda i: (i, 0)),
                  pl.BlockSpec((512, _TOPK), lambda i: (i, 0)),
                  pl.BlockSpec((512, H), lambda i: (i, 0))],
        out_specs=pl.BlockSpec((512, H), lambda i: (i, 0)),
        out_shape=jax.ShapeDtypeStruct((N, H), jnp.float32),
    )(y0g, y1g, w, shared_g)

    return out2.reshape(orig_shape)


# speedup vs baseline: 1.1603x; 1.0504x over previous
import functools

import jax
import jax.numpy as jnp
from jax.experimental import pallas as pl
from jax.experimental.pallas import tpu as pltpu
from jax.experimental.pallas import tpu_sc as plsc

_E = 8
_TOPK = 2
_TM = 128
_RT = 512
_SM = 1024
_SK = 256

_NC = 2
_NS = 16
_NW = _NC * _NS


def _silu(x):
    return x / (1.0 + jnp.exp(-x))


def _router_body(x_ref, rw_ref, w_ref, idx_ref):
    x = x_ref[...]
    rw = rw_ref[...]
    logits = jax.lax.dot_general(x, rw, (((1,), (1,)), ((), ())),
                                 preferred_element_type=jnp.float32)
    m = jnp.max(logits, axis=-1, keepdims=True)
    p = jnp.exp(logits - m)
    p = p / jnp.sum(p, axis=-1, keepdims=True)
    iota = jax.lax.broadcasted_iota(jnp.int32, p.shape, 1)
    m1 = jnp.max(p, axis=-1, keepdims=True)
    i1 = jnp.min(jnp.where(p == m1, iota, _E), axis=-1, keepdims=True)
    pm = jnp.where(iota == i1, -1.0, p)
    m2 = jnp.max(pm, axis=-1, keepdims=True)
    i2 = jnp.min(jnp.where(pm == m2, iota, _E), axis=-1, keepdims=True)
    s = jnp.maximum(m1 + m2, 1e-9)
    w_ref[...] = jnp.concatenate([m1 / s, m2 / s], axis=-1)
    idx_ref[...] = jnp.concatenate([i1, i2], axis=-1).astype(jnp.int32)


def _make_sc_dispatch(NA, H, C, M_PAD):
    rows_per_w = NA // _NW
    n_chunks = rows_per_w // C
    mesh = plsc.VectorSubcoreMesh(core_axis_name="c", subcore_axis_name="s")

    @functools.partial(
        pl.kernel,
        out_type=jax.ShapeDtypeStruct((M_PAD, H), jnp.float32),
        mesh=mesh,
        scratch_types=[
            pltpu.VMEM((C,), jnp.int32),
            pltpu.VMEM((C,), jnp.int32),
            pltpu.VMEM((C, H), jnp.float32),
            pltpu.SemaphoreType.DMA,
            pltpu.SemaphoreType.DMA,
        ],
    )
    def k(tok_hbm, pos_hbm, src_hbm, out_hbm, tok_v, pos_v, rows_v, s1, s2):
        wid = jax.lax.axis_index("s") * _NC + jax.lax.axis_index("c")
        base = wid * rows_per_w

        def body(j, carry):
            off = base + j * C
            pltpu.sync_copy(tok_hbm.at[pl.ds(off, C)], tok_v)
            pltpu.sync_copy(pos_hbm.at[pl.ds(off, C)], pos_v)
            pltpu.async_copy(src_hbm.at[tok_v], rows_v, s1).wait()
            pltpu.async_copy(rows_v, out_hbm.at[pos_v], s2).wait()
            return carry

        jax.lax.fori_loop(0, n_chunks, body, 0)

    return k


def _make_sc_combine_gather(N, H, C):
    rows_per_w = N // _NW
    n_chunks = rows_per_w // C
    mesh = plsc.VectorSubcoreMesh(core_axis_name="c", subcore_axis_name="s")

    @functools.partial(
        pl.kernel,
        out_type=[jax.ShapeDtypeStruct((N, H), jnp.float32),
                  jax.ShapeDtypeStruct((N, H), jnp.float32)],
        mesh=mesh,
        scratch_types=[
            pltpu.VMEM((C,), jnp.int32),
            pltpu.VMEM((C, H), jnp.float32),
            pltpu.SemaphoreType.DMA,
        ],
    )
    def k(p0_hbm, p1_hbm, y_hbm, o0_hbm, o1_hbm, idx_v, rows_v, sem):
        wid = jax.lax.axis_index("s") * _NC + jax.lax.axis_index("c")
        base = wid * rows_per_w

        def body(j, carry):
            off = base + j * C
            pltpu.sync_copy(p0_hbm.at[pl.ds(off, C)], idx_v)
            pltpu.async_copy(y_hbm.at[idx_v], rows_v, sem).wait()
            pltpu.sync_copy(rows_v, o0_hbm.at[pl.ds(off, C)])
            pltpu.sync_copy(p1_hbm.at[pl.ds(off, C)], idx_v)
            pltpu.async_copy(y_hbm.at[idx_v], rows_v, sem).wait()
            pltpu.sync_copy(rows_v, o1_hbm.at[pl.ds(off, C)])
            return carry

        jax.lax.fori_loop(0, n_chunks, body, 0)

    return k


def _gmm_body(e_ref, x_ref, wg_ref, wu_ref, wd_ref, y_ref):
    del e_ref
    x = x_ref[...]
    g = jax.lax.dot_general(x, wg_ref[0], (((1,), (1,)), ((), ())),
                            preferred_element_type=jnp.float32)
    u = jax.lax.dot_general(x, wu_ref[0], (((1,), (1,)), ((), ())),
                            preferred_element_type=jnp.float32)
    a = _silu(g) * u
    y_ref[...] = jax.lax.dot_general(a, wd_ref[0], (((1,), (1,)), ((), ())),
                                     preferred_element_type=jnp.float32)


def _shared_body(n_k, x_ref, sg_ref, su_ref, sd_ref, gw_ref, o_ref):
    k = pl.program_id(1)
    x = x_ref[...]
    g = jax.lax.dot_general(x, sg_ref[...], (((1,), (1,)), ((), ())),
                            preferred_element_type=jnp.float32)
    u = jax.lax.dot_general(x, su_ref[...], (((1,), (1,)), ((), ())),
                            preferred_element_type=jnp.float32)
    a = _silu(g) * u
    part = jax.lax.dot_general(a, sd_ref[...], (((1,), (1,)), ((), ())),
                               preferred_element_type=jnp.float32)

    @pl.when(k == 0)
    def _():
        o_ref[...] = part

    @pl.when(k != 0)
    def _():
        o_ref[...] += part

    @pl.when(k == n_k - 1)
    def _():
        gl = jax.lax.dot_general(x_ref[...], gw_ref[...],
                                 (((1,), (1,)), ((), ())),
                                 preferred_element_type=jnp.float32)
        o_ref[...] *= 1.0 / (1.0 + jnp.exp(-gl))


def _combine_body(y0_ref, y1_ref, w_ref, sh_ref, o_ref):
    w = w_ref[...]
    o_ref[...] = (y0_ref[...] * w[:, 0:1] + y1_ref[...] * w[:, 1:2]
                  + sh_ref[...])


def kernel(hidden_states, router_w, Wg, Wu, Wd, Sg, Su, Sd, gate_w):
    orig_shape = hidden_states.shape
    H = orig_shape[-1]
    h2 = hidden_states.reshape(-1, H)
    N = h2.shape[0]
    INTER = Wg.shape[1]
    SH = Sg.shape[0]
    NA = N * _TOPK
    M_PAD = NA + _E * _TM
    n_tiles = M_PAD // _TM

    w, idx = pl.pallas_call(
        _router_body,
        grid=(N // _RT,),
        in_specs=[pl.BlockSpec((_RT, H), lambda i: (i, 0)),
                  pl.BlockSpec((_E, H), lambda i: (0, 0))],
        out_specs=[pl.BlockSpec((_RT, _TOPK), lambda i: (i, 0)),
                   pl.BlockSpec((_RT, _TOPK), lambda i: (i, 0))],
        out_shape=[jax.ShapeDtypeStruct((N, _TOPK), jnp.float32),
                   jax.ShapeDtypeStruct((N, _TOPK), jnp.int32)],
    )(h2, router_w)

    a = idx.reshape(-1)
    oh = (a[:, None] == jnp.arange(_E, dtype=jnp.int32)[None, :]).astype(jnp.int32)
    ranks = jnp.cumsum(oh, axis=0) - oh
    counts = jnp.sum(oh, axis=0)
    padded = ((counts + _TM - 1) // _TM) * _TM
    base = jnp.concatenate([jnp.zeros((1,), jnp.int32),
                            jnp.cumsum(padded)[:-1].astype(jnp.int32)])
    pos_flat = base[a] + jnp.take_along_axis(ranks, a[:, None], axis=1)[:, 0]
    tile_start = jnp.arange(n_tiles, dtype=jnp.int32) * _TM
    expert_of_tile = jnp.clip(
        jnp.searchsorted(base, tile_start, side="right") - 1, 0, _E - 1
    ).astype(jnp.int32)
    pos = pos_flat.reshape(N, _TOPK)
    tok_src = jnp.arange(NA, dtype=jnp.int32) // _TOPK

    x_sorted = _make_sc_dispatch(NA, H, 16, M_PAD)(tok_src, pos_flat, h2)

    n_k = SH // _SK
    shared_g = pl.pallas_call(
        functools.partial(_shared_body, n_k),
        grid=(N // _SM, n_k),
        in_specs=[
            pl.BlockSpec((_SM, H), lambda i, k: (i, 0)),
            pl.BlockSpec((_SK, H), lambda i, k: (k, 0)),
            pl.BlockSpec((_SK, H), lambda i, k: (k, 0)),
            pl.BlockSpec((H, _SK), lambda i, k: (0, k)),
            pl.BlockSpec((1, H), lambda i, k: (0, 0)),
        ],
        out_specs=pl.BlockSpec((_SM, H), lambda i, k: (i, 0)),
        out_shape=jax.ShapeDtypeStruct((N, H), jnp.float32),
    )(h2, Sg, Su, Sd, gate_w)

    y_sorted = pl.pallas_call(
        _gmm_body,
        grid_spec=pltpu.PrefetchScalarGridSpec(
            num_scalar_prefetch=1,
            grid=(n_tiles,),
            in_specs=[
                pl.BlockSpec((_TM, H), lambda i, e: (i, 0)),
                pl.BlockSpec((1, INTER, H), lambda i, e: (e[i], 0, 0)),
                pl.BlockSpec((1, INTER, H), lambda i, e: (e[i], 0, 0)),
                pl.BlockSpec((1, H, INTER), lambda i, e: (e[i], 0, 0)),
            ],
            out_specs=pl.BlockSpec((_TM, H), lambda i, e: (i, 0)),
        ),
        out_shape=jax.ShapeDtypeStruct((M_PAD, H), jnp.float32),
    )(expert_of_tile, x_sorted, Wg, Wu, Wd)

    y0g, y1g = _make_sc_combine_gather(N, H, 16)(pos[:, 0], pos[:, 1],
                                                 y_sorted)
    out2 = pl.pallas_call(
        _combine_body,
        grid=(N // 512,),
        in_specs=[pl.BlockSpec((512, H), lambda i: (i, 0)),
                  pl.BlockSpec((512, H), lambda i: (i, 0)),
                  pl.BlockSpec((512, _TOPK), lambda i: (i, 0)),
                  pl.BlockSpec((512, H), lambda i: (i, 0))],
        out_specs=pl.BlockSpec((512, H), lambda i: (i, 0)),
        out_shape=jax.ShapeDtypeStruct((N, H), jnp.float32),
    )(y0g, y1g, w, shared_g)

    return out2.reshape(orig_shape)


# speedup vs baseline: 1.4430x; 1.2436x over previous
import functools

import jax
import jax.numpy as jnp
from jax.experimental import pallas as pl
from jax.experimental.pallas import tpu as pltpu
from jax.experimental.pallas import tpu_sc as plsc

_E = 8
_TOPK = 2
_TM = 256
_RT = 512
_SM = 1024
_SK = 256

_NC = 2
_NS = 16
_NW = _NC * _NS


def _silu(x):
    return x / (1.0 + jnp.exp(-x))


def _router_body(x_ref, rw_ref, w_ref, idx_ref):
    x = x_ref[...]
    rw = rw_ref[...]
    logits = jax.lax.dot_general(x, rw, (((1,), (1,)), ((), ())),
                                 preferred_element_type=jnp.float32)
    m = jnp.max(logits, axis=-1, keepdims=True)
    p = jnp.exp(logits - m)
    p = p / jnp.sum(p, axis=-1, keepdims=True)
    iota = jax.lax.broadcasted_iota(jnp.int32, p.shape, 1)
    m1 = jnp.max(p, axis=-1, keepdims=True)
    i1 = jnp.min(jnp.where(p == m1, iota, _E), axis=-1, keepdims=True)
    pm = jnp.where(iota == i1, -1.0, p)
    m2 = jnp.max(pm, axis=-1, keepdims=True)
    i2 = jnp.min(jnp.where(pm == m2, iota, _E), axis=-1, keepdims=True)
    s = jnp.maximum(m1 + m2, 1e-9)
    w_ref[...] = jnp.concatenate([m1 / s, m2 / s], axis=-1)
    idx_ref[...] = jnp.concatenate([i1, i2], axis=-1).astype(jnp.int32)


def _make_sc_dispatch(NA, H, C, M_PAD):
    rows_per_w = NA // _NW
    n_chunks = rows_per_w // C
    mesh = plsc.VectorSubcoreMesh(core_axis_name="c", subcore_axis_name="s")

    @functools.partial(
        pl.kernel,
        out_type=jax.ShapeDtypeStruct((M_PAD, H), jnp.float32),
        mesh=mesh,
        scratch_types=[
            pltpu.VMEM((C,), jnp.int32),
            pltpu.VMEM((C,), jnp.int32),
            pltpu.VMEM((C, H), jnp.float32),
            pltpu.SemaphoreType.DMA,
            pltpu.SemaphoreType.DMA,
        ],
    )
    def k(tok_hbm, pos_hbm, src_hbm, out_hbm, tok_v, pos_v, rows_v, s1, s2):
        wid = jax.lax.axis_index("s") * _NC + jax.lax.axis_index("c")
        base = wid * rows_per_w

        def body(j, carry):
            off = base + j * C
            pltpu.sync_copy(tok_hbm.at[pl.ds(off, C)], tok_v)
            pltpu.sync_copy(pos_hbm.at[pl.ds(off, C)], pos_v)
            pltpu.async_copy(src_hbm.at[tok_v], rows_v, s1).wait()
            pltpu.async_copy(rows_v, out_hbm.at[pos_v], s2).wait()
            return carry

        jax.lax.fori_loop(0, n_chunks, body, 0)

    return k


def _make_sc_combine_gather(N, H, C):
    rows_per_w = N // _NW
    n_chunks = rows_per_w // C
    mesh = plsc.VectorSubcoreMesh(core_axis_name="c", subcore_axis_name="s")

    @functools.partial(
        pl.kernel,
        out_type=[jax.ShapeDtypeStruct((N, H), jnp.float32),
                  jax.ShapeDtypeStruct((N, H), jnp.float32)],
        mesh=mesh,
        scratch_types=[
            pltpu.VMEM((C,), jnp.int32),
            pltpu.VMEM((C, H), jnp.float32),
            pltpu.SemaphoreType.DMA,
        ],
    )
    def k(p0_hbm, p1_hbm, y_hbm, o0_hbm, o1_hbm, idx_v, rows_v, sem):
        wid = jax.lax.axis_index("s") * _NC + jax.lax.axis_index("c")
        base = wid * rows_per_w

        def body(j, carry):
            off = base + j * C
            pltpu.sync_copy(p0_hbm.at[pl.ds(off, C)], idx_v)
            pltpu.async_copy(y_hbm.at[idx_v], rows_v, sem).wait()
            pltpu.sync_copy(rows_v, o0_hbm.at[pl.ds(off, C)])
            pltpu.sync_copy(p1_hbm.at[pl.ds(off, C)], idx_v)
            pltpu.async_copy(y_hbm.at[idx_v], rows_v, sem).wait()
            pltpu.sync_copy(rows_v, o1_hbm.at[pl.ds(off, C)])
            return carry

        jax.lax.fori_loop(0, n_chunks, body, 0)

    return k


def _gmm_body(e_ref, x_ref, wg_ref, wu_ref, wd_ref, y_ref):
    del e_ref
    x = x_ref[...]
    g = jax.lax.dot_general(x, wg_ref[0], (((1,), (1,)), ((), ())),
                            preferred_element_type=jnp.float32)
    u = jax.lax.dot_general(x, wu_ref[0], (((1,), (1,)), ((), ())),
                            preferred_element_type=jnp.float32)
    a = _silu(g) * u
    y_ref[...] = jax.lax.dot_general(a, wd_ref[0], (((1,), (1,)), ((), ())),
                                     preferred_element_type=jnp.float32)


def _shared_body(n_k, x_ref, sg_ref, su_ref, sd_ref, gw_ref, o_ref):
    k = pl.program_id(1)
    x = x_ref[...]
    g = jax.lax.dot_general(x, sg_ref[...], (((1,), (1,)), ((), ())),
                            preferred_element_type=jnp.float32)
    u = jax.lax.dot_general(x, su_ref[...], (((1,), (1,)), ((), ())),
                            preferred_element_type=jnp.float32)
    a = _silu(g) * u
    part = jax.lax.dot_general(a, sd_ref[...], (((1,), (1,)), ((), ())),
                               preferred_element_type=jnp.float32)

    @pl.when(k == 0)
    def _():
        o_ref[...] = part

    @pl.when(k != 0)
    def _():
        o_ref[...] += part

    @pl.when(k == n_k - 1)
    def _():
        gl = jax.lax.dot_general(x_ref[...], gw_ref[...],
                                 (((1,), (1,)), ((), ())),
                                 preferred_element_type=jnp.float32)
        o_ref[...] *= 1.0 / (1.0 + jnp.exp(-gl))


def _combine_body(y0_ref, y1_ref, w_ref, sh_ref, o_ref):
    w = w_ref[...]
    o_ref[...] = (y0_ref[...] * w[:, 0:1] + y1_ref[...] * w[:, 1:2]
                  + sh_ref[...])


def kernel(hidden_states, router_w, Wg, Wu, Wd, Sg, Su, Sd, gate_w):
    orig_shape = hidden_states.shape
    H = orig_shape[-1]
    h2 = hidden_states.reshape(-1, H)
    N = h2.shape[0]
    INTER = Wg.shape[1]
    SH = Sg.shape[0]
    NA = N * _TOPK
    M_PAD = NA + _E * _TM
    n_tiles = M_PAD // _TM

    w, idx = pl.pallas_call(
        _router_body,
        grid=(N // _RT,),
        in_specs=[pl.BlockSpec((_RT, H), lambda i: (i, 0)),
                  pl.BlockSpec((_E, H), lambda i: (0, 0))],
        out_specs=[pl.BlockSpec((_RT, _TOPK), lambda i: (i, 0)),
                   pl.BlockSpec((_RT, _TOPK), lambda i: (i, 0))],
        out_shape=[jax.ShapeDtypeStruct((N, _TOPK), jnp.float32),
                   jax.ShapeDtypeStruct((N, _TOPK), jnp.int32)],
    )(h2, router_w)

    a = idx.reshape(-1)
    oh = (a[:, None] == jnp.arange(_E, dtype=jnp.int32)[None, :]).astype(jnp.int32)
    ranks = jnp.cumsum(oh, axis=0) - oh
    counts = jnp.sum(oh, axis=0)
    padded = ((counts + _TM - 1) // _TM) * _TM
    base = jnp.concatenate([jnp.zeros((1,), jnp.int32),
                            jnp.cumsum(padded)[:-1].astype(jnp.int32)])
    pos_flat = base[a] + jnp.take_along_axis(ranks, a[:, None], axis=1)[:, 0]
    tile_start = jnp.arange(n_tiles, dtype=jnp.int32) * _TM
    expert_of_tile = jnp.clip(
        jnp.searchsorted(base, tile_start, side="right") - 1, 0, _E - 1
    ).astype(jnp.int32)
    pos = pos_flat.reshape(N, _TOPK)
    tok_src = jnp.arange(NA, dtype=jnp.int32) // _TOPK

    x_sorted = _make_sc_dispatch(NA, H, 16, M_PAD)(tok_src, pos_flat, h2)

    n_k = SH // _SK
    shared_g = pl.pallas_call(
        functools.partial(_shared_body, n_k),
        grid=(N // _SM, n_k),
        in_specs=[
            pl.BlockSpec((_SM, H), lambda i, k: (i, 0)),
            pl.BlockSpec((_SK, H), lambda i, k: (k, 0)),
            pl.BlockSpec((_SK, H), lambda i, k: (k, 0)),
            pl.BlockSpec((H, _SK), lambda i, k: (0, k)),
            pl.BlockSpec((1, H), lambda i, k: (0, 0)),
        ],
        out_specs=pl.BlockSpec((_SM, H), lambda i, k: (i, 0)),
        out_shape=jax.ShapeDtypeStruct((N, H), jnp.float32),
    )(h2, Sg, Su, Sd, gate_w)

    y_sorted = pl.pallas_call(
        _gmm_body,
        grid_spec=pltpu.PrefetchScalarGridSpec(
            num_scalar_prefetch=1,
            grid=(n_tiles,),
            in_specs=[
                pl.BlockSpec((_TM, H), lambda i, e: (i, 0)),
                pl.BlockSpec((1, INTER, H), lambda i, e: (e[i], 0, 0)),
                pl.BlockSpec((1, INTER, H), lambda i, e: (e[i], 0, 0)),
                pl.BlockSpec((1, H, INTER), lambda i, e: (e[i], 0, 0)),
            ],
            out_specs=pl.BlockSpec((_TM, H), lambda i, e: (i, 0)),
        ),
        out_shape=jax.ShapeDtypeStruct((M_PAD, H), jnp.float32),
    )(expert_of_tile, x_sorted, Wg, Wu, Wd)

    y0g, y1g = _make_sc_combine_gather(N, H, 16)(pos[:, 0], pos[:, 1],
                                                 y_sorted)
    out2 = pl.pallas_call(
        _combine_body,
        grid=(N // 512,),
        in_specs=[pl.BlockSpec((512, H), lambda i: (i, 0)),
                  pl.BlockSpec((512, H), lambda i: (i, 0)),
                  pl.BlockSpec((512, _TOPK), lambda i: (i, 0)),
                  pl.BlockSpec((512, H), lambda i: (i, 0))],
        out_specs=pl.BlockSpec((512, H), lambda i: (i, 0)),
        out_shape=jax.ShapeDtypeStruct((N, H), jnp.float32),
    )(y0g, y1g, w, shared_g)

    return out2.reshape(orig_shape)
